# Initial kernel scaffold; baseline (speedup 1.0000x reference)
import jax
import jax.numpy as jnp
from jax.experimental import pallas as pl


def kernel(edge_index, edge_attr, params):
    raise NotImplementedError("write your pallas kernel here")


# speedup vs baseline: 2.6543x; 2.6543x over previous
import functools
import jax
import jax.numpy as jnp
from jax import lax
from jax.experimental import pallas as pl
from jax.experimental.pallas import tpu as pltpu
from jax.experimental.pallas import tpu_sc as plsc

V = 4096
E = 131072
N_PAIRS = V * (V - 1) // 2
HALF = N_PAIRS // 2
REG = 1 << 22
NPAD = 2 * REG
NC, NS = 2, 16
NW = NC * NS
ET = E // NW
CHW = 128
CH = ET // CHW

_mesh = plsc.VectorSubcoreMesh(
    core_axis_name="c", subcore_axis_name="s", num_cores=NC, num_subcores=NS)
_sc_params = pltpu.CompilerParams(use_tc_tiling_on_sc=False)


def _zero_vec(n):
  return jnp.zeros((n,), jnp.float32)


@functools.partial(
    pl.kernel,
    out_type=jax.ShapeDtypeStruct((NC * V,), jnp.float32),
    mesh=_mesh,
    scratch_types=[
        pltpu.VMEM((CH, CHW), jnp.int32),
        pltpu.VMEM((CHW,), jnp.float32),
        pltpu.VMEM((V // NS,), jnp.float32),
        pltpu.VMEM_SHARED((V,), jnp.float32),
    ],
    compiler_params=_sc_params,
)
def _sc_deg(dst2d, out, dstb, ones_v, zb, acc):
  cid = lax.axis_index("c")
  sid = lax.axis_index("s")
  sl = V // NS
  for i in range(sl // 16):
    zb[pl.ds(i * 16, 16)] = _zero_vec(16)
  for i in range(CHW // 16):
    ones_v[pl.ds(i * 16, 16)] = jnp.ones((16,), jnp.float32)
  pltpu.sync_copy(zb, acc.at[pl.ds(sid * sl, sl)])
  plsc.subcore_barrier()
  rowbase = (cid * NS + sid) * CH
  pltpu.sync_copy(dst2d.at[pl.ds(rowbase, CH), :], dstb)
  for ch in range(CH):
    pltpu.sync_copy(ones_v, acc.at[dstb.at[ch]], add=True)
  plsc.subcore_barrier()
  pltpu.sync_copy(acc.at[pl.ds(sid * sl, sl)],
                  out.at[pl.ds(cid * V + sid * sl, sl)])


def _make_sc_gather(co_pq, co_acc):
  outs = []
  scratch = [pltpu.VMEM((CH, CHW), jnp.int32),
             pltpu.VMEM((CH, CHW), jnp.int32)]
  if co_pq:
    outs += [jax.ShapeDtypeStruct((E, co_pq), jnp.float32),
             jax.ShapeDtypeStruct((E, co_pq), jnp.float32)]
    scratch += [pltpu.VMEM((CHW, co_pq), jnp.float32),
                pltpu.VMEM((CHW, co_pq), jnp.float32)]
  if co_acc:
    outs += [jax.ShapeDtypeStruct((NC * V, co_acc), jnp.float32)]
    scratch += [pltpu.VMEM((CHW, co_acc), jnp.float32),
                pltpu.VMEM((16, co_acc), jnp.float32),
                pltpu.VMEM_SHARED((V, co_acc), jnp.float32)]
  scratch += [pltpu.SemaphoreType.DMA, pltpu.SemaphoreType.DMA,
              pltpu.SemaphoreType.DMA]

  @functools.partial(pl.kernel, out_type=tuple(outs), mesh=_mesh,
                     scratch_types=scratch, compiler_params=_sc_params)
  def k(*refs):
    i = 0
    src2d, dst2d = refs[i], refs[i + 1]; i += 2
    if co_pq:
      ptab, qtab = refs[i], refs[i + 1]; i += 2
    if co_acc:
      htab = refs[i]; i += 1
    if co_pq:
      pout, qout = refs[i], refs[i + 1]; i += 2
    if co_acc:
      aout = refs[i]; i += 1
    srcb, dstb = refs[i], refs[i + 1]; i += 2
    if co_pq:
      prow, qrow = refs[i], refs[i + 1]; i += 2
    if co_acc:
      hrow, zb, acc = refs[i], refs[i + 1], refs[i + 2]; i += 3
    sem0, sem1, sem2 = refs[i], refs[i + 1], refs[i + 2]

    cid = lax.axis_index("c")
    sid = lax.axis_index("s")
    if co_acc:
      for r in range(16):
        for j in range(co_acc // 16):
          zb[r, pl.ds(j * 16, 16)] = _zero_vec(16)
      sl = V // NS
      for kk in range(sl // 16):
        pltpu.sync_copy(zb, acc.at[pl.ds(sid * sl + kk * 16, 16), :])
      plsc.subcore_barrier()
    rowbase = (cid * NS + sid) * CH
    pltpu.sync_copy(src2d.at[pl.ds(rowbase, CH), :], srcb)
    pltpu.sync_copy(dst2d.at[pl.ds(rowbase, CH), :], dstb)
    ebase = (cid * NS + sid) * ET
    for ch in range(CH):
      d = []
      if co_pq:
        d.append(pltpu.async_copy(ptab.at[srcb.at[ch]], prow, sem0))
        d.append(pltpu.async_copy(qtab.at[dstb.at[ch]], qrow, sem1))
      if co_acc:
        d.append(pltpu.async_copy(htab.at[srcb.at[ch]], hrow, sem2))
      for dd in d:
        dd.wait()
      if co_pq:
        pltpu.sync_copy(prow, pout.at[pl.ds(ebase + ch * CHW, CHW), :])
        pltpu.sync_copy(qrow, qout.at[pl.ds(ebase + ch * CHW, CHW), :])
      if co_acc:
        pltpu.sync_copy(hrow, acc.at[dstb.at[ch]], add=True)
    if co_acc:
      plsc.subcore_barrier()
      sl = V // NS
      pltpu.sync_copy(acc.at[pl.ds(sid * sl, sl), :],
                      aout.at[pl.ds(cid * V + sid * sl, sl), :])

  return k


_sc_gather_acc = _make_sc_gather(0, 128)
_sc_gather_pq_acc = _make_sc_gather(128, 128)
_sc_gather_pq = _make_sc_gather(128, 0)


@functools.partial(
    pl.kernel,
    out_type=jax.ShapeDtypeStruct((NPAD,), jnp.float32),
    mesh=_mesh,
    scratch_types=[
        pltpu.VMEM((4096,), jnp.float32),
        pltpu.VMEM((16,), jnp.float32),
        pltpu.VMEM((E // NS // 128, 128), jnp.int32),
        pltpu.VMEM((E // NS // 128, 128), jnp.int32),
        pltpu.VMEM((E // NS // 128, 128), jnp.float32),
        pltpu.VMEM((E // NS // 128, 128), jnp.int32),
        pltpu.VMEM((E // NS // 128, 128), jnp.float32),
        pltpu.SemaphoreType.DMA,
    ],
    compiler_params=_sc_params,
)
def _sc_policy(src2d, dst2d, s2d, c0h, out, fillb, c0v, srcb, dstb, sb,
               idxb, valb, sem):
  cid = lax.axis_index("c")
  sid = lax.axis_index("s")
  pltpu.sync_copy(c0h, c0v)
  c0 = c0v[...]

  def fill_body(i, _):
    fillb[pl.ds(i * 16, 16)] = c0
    return 0
  lax.fori_loop(0, 256, fill_body, 0)
  tbase = cid * REG + sid * (REG // NS)
  for kk in range(REG // NS // 4096):
    pltpu.sync_copy(fillb, out.at[pl.ds(tbase + kk * 4096, 4096)])
  plsc.subcore_barrier()

  nrows = E // NS // 128
  rowbase = sid * nrows
  pltpu.sync_copy(src2d.at[pl.ds(rowbase, nrows), :], srcb)
  pltpu.sync_copy(dst2d.at[pl.ds(rowbase, nrows), :], dstb)
  pltpu.sync_copy(s2d.at[pl.ds(rowbase, nrows), :], sb)
  lane = lax.iota(jnp.int32, 16)
  half_lo = cid * HALF
  shift = cid * (REG - HALF)
  dump0 = cid * REG + HALF

  def row_body(r, _):
    for j in range(8):
      srcv = srcb[r, pl.ds(j * 16, 16)]
      dstv = dstb[r, pl.ds(j * 16, 16)]
      sv = sb[r, pl.ds(j * 16, 16)]
      pair = (srcv * V - ((srcv * (srcv + 1)) >> 1) + dstv - srcv - 1)
      mask = (srcv < dstv) & (pair >= half_lo) & (pair < half_lo + HALF)
      eloc = (rowbase + r) * 128 + j * 16 + lane
      tgt = jnp.where(mask, pair + shift, dump0 + (eloc & 1023))
      idxb[r, pl.ds(j * 16, 16)] = tgt
      valb[r, pl.ds(j * 16, 16)] = c0 * jnp.exp(sv)
    return 0
  lax.fori_loop(0, nrows, row_body, 0)
  descs = []
  for r in range(nrows):
    descs.append(pltpu.async_copy(valb.at[r], out.at[idxb.at[r]], sem))
  for d in descs:
    d.wait()


def _vmem_call(body, out_shapes, n_in):
  return pl.pallas_call(
      body,
      out_shape=out_shapes,
      in_specs=[pl.BlockSpec(memory_space=pltpu.ANY if False else pltpu.VMEM)
                for _ in range(n_in)],
      out_specs=jax.tree.map(
          lambda _: pl.BlockSpec(memory_space=pltpu.VMEM), out_shapes),
  )


def _tc_prep_body(degp, emb, wg, dinv_o, hs_o):
  d = degp[0] + degp[1] + 1.0
  dinv = lax.rsqrt(d)
  dinv_o[...] = dinv
  hs_o[...] = jnp.zeros_like(hs_o)
  hs_o[:, 0:64] = jnp.dot(emb[...], wg[...],
                          preferred_element_type=jnp.float32) * dinv


def _tc_node_body(is_last, co, accp, hs, dinv, bg, g, beta, w1b, w1ab, b1,
                  wnext, *rest):
  if is_last:
    vw1, vb1, vw2, vb2, vw3, vb3 = rest[:6]
    p_o, q_o, v_o = rest[6:]
  else:
    p_o, q_o, hsn_o = rest
  acc = (accp[0] + accp[1] + hs[...])[:, 0:co]
  xo = dinv[...] * acc + bg[...]
  mu = jnp.mean(xo, axis=0, keepdims=True)
  var = jnp.mean((xo - mu) * (xo - mu), axis=0, keepdims=True)
  xn = jnp.maximum((xo - mu) * lax.rsqrt(var + 1e-5) * g[...] + beta[...], 0.0)
  cw = w1b.shape[1]
  if cw < 128:
    p_o[...] = jnp.zeros_like(p_o)
    q_o[...] = jnp.zeros_like(q_o)
  p_o[:, 0:cw] = jnp.dot(xn, w1b[...], preferred_element_type=jnp.float32)
  q_o[:, 0:cw] = jnp.dot(xn, w1ab[...],
                         preferred_element_type=jnp.float32) + b1[...]
  if is_last:
    pooled = jnp.mean(xn, axis=0, keepdims=True)
    v = jnp.maximum(jnp.dot(pooled, vw1[...],
                            preferred_element_type=jnp.float32) + vb1[...], 0.)
    v = jnp.maximum(jnp.dot(v, vw2[...],
                            preferred_element_type=jnp.float32) + vb2[...], 0.)
    v_o[...] = jnp.tanh(jnp.dot(v, vw3[...],
                                preferred_element_type=jnp.float32) + vb3[...])
  else:
    hsn_o[...] = jnp.dot(xn, wnext[...],
                         preferred_element_type=jnp.float32) * dinv[...]


BE = 2048


def _tc_edge_body(first, cw, ps, qd, prev, wc, bc, ab, bb, w2, b2,
                  h_o, sum_o, ssq_o, accs, accq):
  step = pl.program_id(0)
  if first:
    r2 = jnp.dot(prev[...], wc[...], preferred_element_type=jnp.float32) + bc[...]
  else:
    ef = jnp.maximum(prev[...] * ab[...] + bb[...], 0.0)
    r2 = jnp.dot(ef, wc[...], preferred_element_type=jnp.float32)
  z = jnp.maximum(ps[:, 0:cw] + qd[:, 0:cw] + r2, 0.0)
  h = jnp.dot(z, w2[...], preferred_element_type=jnp.float32) + b2[...]
  h_o[...] = h

  @pl.when(step == 0)
  def _():
    accs[...] = jnp.zeros_like(accs)
    accq[...] = jnp.zeros_like(accq)
  accs[0:1] += jnp.sum(h, axis=0, keepdims=True)
  accq[0:1] += jnp.sum(h * h, axis=0, keepdims=True)

  @pl.when(step == pl.num_programs(0) - 1)
  def _():
    sum_o[...] = accs[0:1]
    ssq_o[...] = accq[0:1]


def _tc_score_body(h2, ab, bb, pw1, pb1, pw2, pb2, srcc, dstc,
                   s_o, stats_o, mrun, srun, mcnt):
  step = pl.program_id(0)
  ef = jnp.maximum(h2[...] * ab[...] + bb[...], 0.0)
  t = jnp.maximum(jnp.dot(ef, pw1[...],
                          preferred_element_type=jnp.float32) + pb1[...], 0.0)
  s = jnp.dot(t, pw2[...], preferred_element_type=jnp.float32) + pb2[0, 0]
  s_o[...] = s
  mask = srcc[...] < dstc[...]
  sm = jnp.where(mask, s, -1e30)
  bm = jnp.max(sm)

  @pl.when(step == 0)
  def _():
    mrun[0] = -1e30
    srun[0] = 0.0
    mcnt[0] = 0.0
  mold = mrun[0]
  mnew = jnp.maximum(mold, bm)
  srun[0] = srun[0] * jnp.exp(mold - mnew) + jnp.sum(
      jnp.where(mask, jnp.exp(sm - mnew), 0.0))
  mrun[0] = mnew
  mcnt[0] += jnp.sum(mask.astype(jnp.float32))

  @pl.when(step == pl.num_programs(0) - 1)
  def _():
    stats_o[0] = mrun[0]
    stats_o[1] = srun[0]
    stats_o[2] = mcnt[0]
    stats_o[3] = 0.0


def kernel(edge_index, edge_attr, params):
  p = params
  src = edge_index[0].astype(jnp.int32)
  dst = edge_index[1].astype(jnp.int32)
  src2d = src.reshape(E // 128, 128)
  dst2d = dst.reshape(E // 128, 128)

  co_x = [64, 128, 128]
  w1a = [p[f'em{i}_W1'][:co_x[i]] for i in range(3)]
  w1b = [p[f'em{i}_W1'][co_x[i]:2 * co_x[i]] for i in range(3)]
  w1c = [p[f'em{i}_W1'][2 * co_x[i]:] for i in range(3)]
  w1ab = [w1a[i] - w1b[i] for i in range(3)]
  wc0 = p['edge_W'] @ w1c[0]
  bc0 = (p['edge_b'] @ w1c[0]).reshape(1, -1)
  r1 = lambda a: a.reshape(1, -1)

  degp = _sc_deg(dst2d)
  dinv, hs0 = _vmem_call(
      _tc_prep_body,
      (jax.ShapeDtypeStruct((V, 1), jnp.float32),
       jax.ShapeDtypeStruct((V, 128), jnp.float32)), 3)(
          degp.reshape(2, V, 1), p['node_emb'], p['gcn0_W'])

  (accp0,) = _sc_gather_acc(src2d, dst2d, hs0)
  accp0 = accp0.reshape(2, V, 128)
  p0, q0, hs1 = _vmem_call(
      functools.partial(_tc_node_body, False, 64),
      (jax.ShapeDtypeStruct((V, 128), jnp.float32),
       jax.ShapeDtypeStruct((V, 128), jnp.float32),
       jax.ShapeDtypeStruct((V, 128), jnp.float32)), 10)(
          accp0, hs0, dinv, r1(p['gcn0_b']), r1(p['gcn0_g']),
          r1(p['gcn0_beta']), w1b[0], w1ab[0], r1(p['em0_b1']), p['gcn1_W'])

  ps0, qd0, accp1 = _sc_gather_pq_acc(src2d, dst2d, p0, q0, hs1)
  accp1 = accp1.reshape(2, V, 128)

  grid = E // BE
  eb = lambda co: pl.BlockSpec((BE, co), lambda i: (i, 0))
  wb = lambda a: pl.BlockSpec(a.shape, lambda i: tuple(0 for _ in a.shape))

  def edge_call(first, cw, ps, qd, prev, wc, bc, ab, bb, w2, b2, co):
    return pl.pallas_call(
        functools.partial(_tc_edge_body, first, cw),
        grid=(grid,),
        in_specs=[eb(ps.shape[1]), eb(qd.shape[1]), eb(prev.shape[1]),
                  wb(wc), wb(bc), wb(ab), wb(bb), wb(w2), wb(b2)],
        out_specs=[eb(co),
                   pl.BlockSpec((1, co), lambda i: (0, 0)),
                   pl.BlockSpec((1, co), lambda i: (0, 0))],
        out_shape=[jax.ShapeDtypeStruct((E, co), jnp.float32),
                   jax.ShapeDtypeStruct((1, co), jnp.float32),
                   jax.ShapeDtypeStruct((1, co), jnp.float32)],
        scratch_shapes=[pltpu.VMEM((8, co), jnp.float32),
                        pltpu.VMEM((8, co), jnp.float32)],
        compiler_params=pltpu.CompilerParams(
            dimension_semantics=("arbitrary",)),
    )(ps, qd, prev, wc, bc, ab, bb, w2, b2)

  one = jnp.ones((1, 1), jnp.float32)
  h0, sum0, ssq0 = edge_call(True, 64, ps0, qd0, edge_attr, wc0, bc0, one,
                             one, p['em0_W2'], r1(p['em0_b2']), 64)

  def bn_ab(i, ssum, ssq):
    mu = ssum / E
    var = jnp.maximum(ssq / E - mu * mu, 0.0)
    a = p[f'em{i}_g'].reshape(1, -1) / jnp.sqrt(var + 1e-5)
    b = p[f'em{i}_beta'].reshape(1, -1) - mu * a
    return a, b

  a0, b0 = bn_ab(0, sum0, ssq0)

  p1, q1, hs2 = _vmem_call(
      functools.partial(_tc_node_body, False, 128),
      (jax.ShapeDtypeStruct((V, 128), jnp.float32),
       jax.ShapeDtypeStruct((V, 128), jnp.float32),
       jax.ShapeDtypeStruct((V, 128), jnp.float32)), 10)(
          accp1, hs1, dinv, r1(p['gcn1_b']), r1(p['gcn1_g']),
          r1(p['gcn1_beta']), w1b[1], w1ab[1], r1(p['em1_b1']), p['gcn2_W'])

  ps1, qd1, accp2 = _sc_gather_pq_acc(src2d, dst2d, p1, q1, hs2)
  accp2 = accp2.reshape(2, V, 128)
  h1, sum1, ssq1 = edge_call(False, 128, ps1, qd1, h0, w1c[1], bc0, a0, b0,
                             p['em1_W2'], r1(p['em1_b2']), 128)
  a1, b1v = bn_ab(1, sum1, ssq1)

  p2, q2, v = _vmem_call(
      functools.partial(_tc_node_body, True, 128),
      (jax.ShapeDtypeStruct((V, 128), jnp.float32),
       jax.ShapeDtypeStruct((V, 128), jnp.float32),
       jax.ShapeDtypeStruct((1, 1), jnp.float32)), 16)(
          accp2, hs2, dinv, r1(p['gcn2_b']), r1(p['gcn2_g']),
          r1(p['gcn2_beta']), w1b[2], w1ab[2], r1(p['em2_b1']),
          p['gcn2_W'],
          p['val_W1'], r1(p['val_b1']), p['val_W2'], r1(p['val_b2']),
          p['val_W3'], r1(p['val_b3']))

  ps2, qd2 = _sc_gather_pq(src2d, dst2d, p2, q2)
  h2, sum2, ssq2 = edge_call(False, 128, ps2, qd2, h1, w1c[2], bc0, a1, b1v,
                             p['em2_W2'], r1(p['em2_b2']), 128)
  a2, b2v = bn_ab(2, sum2, ssq2)

  srcc = src.reshape(E, 1)
  dstc = dst.reshape(E, 1)
  s_out, stats = pl.pallas_call(
      _tc_score_body,
      grid=(grid,),
      in_specs=[eb(128), wb(a2), wb(b2v), wb(p['pol_W1']),
                pl.BlockSpec((1, 64), lambda i: (0, 0)),
                wb(p['pol_W2']), pl.BlockSpec((1, 1), lambda i: (0, 0)),
                pl.BlockSpec((BE, 1), lambda i: (i, 0)),
                pl.BlockSpec((BE, 1), lambda i: (i, 0))],
      out_specs=[pl.BlockSpec((BE, 1), lambda i: (i, 0)),
                 pl.BlockSpec(memory_space=pltpu.SMEM)],
      out_shape=[jax.ShapeDtypeStruct((E, 1), jnp.float32),
                 jax.ShapeDtypeStruct((4,), jnp.float32)],
      scratch_shapes=[pltpu.SMEM((1,), jnp.float32),
                      pltpu.SMEM((1,), jnp.float32),
                      pltpu.SMEM((1,), jnp.float32)],
      compiler_params=pltpu.CompilerParams(
          dimension_semantics=("arbitrary",)),
  )(h2, a2, b2v, p['pol_W1'], r1(p['pol_b1']), p['pol_W2'],
    p['pol_b2'].reshape(1, 1), srcc, dstc)

  mrun, ssum, mcount = stats[0], stats[1], stats[2]
  m = jnp.maximum(mrun, 0.0)
  z = (N_PAIRS - mcount) * jnp.exp(-m) + ssum * jnp.exp(mrun - m)
  c0 = jnp.exp(-m) / z
  c0vec = jnp.full((16,), c0, jnp.float32)

  padded = _sc_policy(src2d, dst2d, s_out.reshape(E // 128, 128), c0vec)
  policy = jnp.concatenate([padded[:HALF], padded[REG:REG + HALF]])
  return policy, v
